# baseline (speedup 1.0000x reference)
import jax
import jax.numpy as jnp
from jax.experimental import pallas as pl


def _mm_block(a_ref, v_ref, o_ref):
    acc = jnp.dot(a_ref[...], v_ref[...], preferred_element_type=jnp.float32)

    @pl.when(pl.program_id(1) == 0)
    def _init():
        o_ref[...] = acc

    @pl.when(pl.program_id(1) != 0)
    def _accum():
        o_ref[...] += acc


def kernel(matrix, matrix_batch):
    m, k = matrix.shape
    b, _, f = matrix_batch.shape
    n = b * f
    vectors = jnp.swapaxes(matrix_batch, 0, 1).reshape(k, n)

    bm = 512
    bk = 2048
    out = pl.pallas_call(
        _mm_block,
        grid=(m // bm, k // bk),
        in_specs=[
            pl.BlockSpec((bm, bk), lambda i, j: (i, j)),
            pl.BlockSpec((bk, n), lambda i, j: (j, 0)),
        ],
        out_specs=pl.BlockSpec((bm, n), lambda i, j: (i, 0)),
        out_shape=jax.ShapeDtypeStruct((m, n), jnp.float32),
    )(matrix, vectors)

    return jnp.swapaxes(out.reshape(m, b, f), 0, 1)


# speedup vs baseline: 1.2569x; 1.2569x over previous
import jax
import jax.numpy as jnp
from jax.experimental import pallas as pl


def _mm_block(a1_ref, a2_ref, v_ref, o1_ref, o2_ref):
    v = v_ref[...]
    o1_ref[...] = jnp.dot(a1_ref[...], v, preferred_element_type=jnp.float32)
    o2_ref[...] = jnp.dot(a2_ref[...], v, preferred_element_type=jnp.float32)


def kernel(matrix, matrix_batch):
    m, k = matrix.shape
    b, _, f = matrix_batch.shape
    n = b * f
    vectors = jnp.swapaxes(matrix_batch, 0, 1).reshape(k, n)

    bm = 256
    half = m // 2
    steps = half // bm
    o1, o2 = pl.pallas_call(
        _mm_block,
        grid=(steps,),
        in_specs=[
            pl.BlockSpec((bm, k), lambda i: (i, 0)),
            pl.BlockSpec((bm, k), lambda i: (i + steps, 0)),
            pl.BlockSpec((k, n), lambda i: (0, 0)),
        ],
        out_specs=[
            pl.BlockSpec((bm, n), lambda i: (i, 0)),
            pl.BlockSpec((bm, n), lambda i: (i, 0)),
        ],
        out_shape=[
            jax.ShapeDtypeStruct((half, n), jnp.float32),
            jax.ShapeDtypeStruct((half, n), jnp.float32),
        ],
    )(matrix, matrix, vectors)

    out = jnp.concatenate([o1, o2], axis=0)
    return jnp.swapaxes(out.reshape(m, b, f), 0, 1)
